# baseline (speedup 1.0000x reference)
import functools

import jax
import jax.numpy as jnp
import numpy as np
from jax.experimental import pallas as pl
from jax.experimental.pallas import tpu as pltpu


def _yolo_kernel(p_ref, t_ref, out_ref, *, lambda_coord, lambda_noobj,
                 n_b, n_halves, unroll):
    cn = p_ref.shape[1]
    s2n = p_ref.shape[2]
    blk = p_ref.shape[3]
    base = jax.lax.rem(pl.program_id(0), n_halves) * blk
    limit = n_b - base
    iota = jax.lax.broadcasted_iota(jnp.int32, (s2n, 128), 1)

    def lt_body(lt, carry):
        a_loc, a_co, a_cn = carry
        l0 = pl.multiple_of(lt * 128, 128)
        valid = iota < limit - l0

        px, py, pw, ph, pc = (
            p_ref[0, c, :, pl.ds(l0, 128)] for c in range(cn))
        tx, ty, tw, th, tc = (
            t_ref[0, c, :, pl.ds(l0, 128)] for c in range(cn))

        dx = px - tx
        dy = py - ty
        dw = pw - tw
        dh = ph - th
        loc = dx * dx + dy * dy + dw * dw + dh * dh

        phw = pw * 0.5
        phh = ph * 0.5
        thw = tw * 0.5
        thh = th * 0.5
        iw = jnp.maximum(
            jnp.minimum(px + phw, tx + thw)
            - jnp.maximum(px - phw, tx - thw), 0.0)
        ih = jnp.maximum(
            jnp.minimum(py + phh, ty + thh)
            - jnp.maximum(py - phh, ty - thh), 0.0)
        inter = iw * ih
        union = pw * ph + tw * th - inter
        iou = inter / (union + 1e-6)

        dob = iou - pc
        dno = pc - tc
        obj = valid & (tc > 0.0)
        nob = valid & (tc == 0.0)

        return (a_loc + jnp.where(obj, loc, 0.0),
                a_co + jnp.where(obj, dob * dob, 0.0),
                a_cn + jnp.where(nob, dno * dno, 0.0))

    z = jnp.zeros((s2n, 128), jnp.float32)
    a_loc, a_co, a_cn = jax.lax.fori_loop(0, blk // 128, lt_body, (z, z, z),
                                          unroll=unroll)

    loc = lambda_coord * jnp.sum(a_loc)
    co = jnp.sum(a_co)
    cn_ = lambda_noobj * jnp.sum(a_cn)
    tot = loc + co + cn_
    ol = jax.lax.broadcasted_iota(jnp.int32, (8, 128), 1)
    out_ref[0] = jnp.where(ol == 0, tot,
                 jnp.where(ol == 1, loc,
                 jnp.where(ol == 2, co,
                 jnp.where(ol == 3, cn_, 0.0))))


@functools.partial(jax.jit, static_argnames=("lambda_coord", "lambda_noobj"))
def _yolo_v1_loss(pred, target, lambda_coord=5.0, lambda_noobj=0.5):
    assert pred.shape == target.shape and pred.shape[-1] == 5
    b, s1, s2, c = pred.shape

    pt = jnp.transpose(pred, (1, 3, 2, 0))
    tt = jnp.transpose(target, (1, 3, 2, 0))

    blk = (-(-b // 2) + 127) // 128 * 128
    n_halves = 2 if blk < b else 1
    if n_halves == 1:
        blk = -(-b // 128) * 128

    itemsize = np.dtype(pred.dtype).itemsize
    cost = pl.CostEstimate(
        flops=64 * b * s1 * s2 * c,
        transcendentals=0,
        bytes_accessed=2 * b * s1 * s2 * c * itemsize + s1 * 8 * 128 * 4)

    body = functools.partial(
        _yolo_kernel,
        lambda_coord=float(lambda_coord),
        lambda_noobj=float(lambda_noobj),
        n_b=b, n_halves=n_halves, unroll=4)

    grid = (s1 * n_halves,)
    out = pl.pallas_call(
        body,
        out_shape=jax.ShapeDtypeStruct((grid[0], 8, 128), jnp.float32),
        grid=grid,
        in_specs=[
            pl.BlockSpec((1, c, s2, blk),
                         lambda k, nh=n_halves: (k // nh, 0, 0, k % nh)),
            pl.BlockSpec((1, c, s2, blk),
                         lambda k, nh=n_halves: (k // nh, 0, 0, k % nh)),
        ],
        out_specs=pl.BlockSpec((1, 8, 128), lambda k: (k, 0, 0)),
        compiler_params=pltpu.CompilerParams(
            dimension_semantics=("parallel",),
            vmem_limit_bytes=48 * 1024 * 1024),
        cost_estimate=cost,
    )(pt, tt)

    totals = jnp.sum(out[:, 0, :4], axis=0)
    return totals[0], totals[1], totals[2], totals[3]


def kernel(pred, target):
    return _yolo_v1_loss(pred, target)


# speedup vs baseline: 1.0810x; 1.0810x over previous
import functools

import jax
import jax.numpy as jnp
import numpy as np
from jax.experimental import pallas as pl
from jax.experimental.pallas import tpu as pltpu


def _yolo_kernel(p_ref, t_ref, out_ref, *, lambda_coord, lambda_noobj,
                 n_b, unroll):
    cn = p_ref.shape[1]
    s2n = p_ref.shape[2]
    n_full = n_b // 128
    n_tail = n_b - n_full * 128

    def cell_terms(l0, width):
        px, py, pw, ph, pc = (
            p_ref[0, c, :, pl.ds(l0, width)] for c in range(cn))
        tx, ty, tw, th, tc = (
            t_ref[0, c, :, pl.ds(l0, width)] for c in range(cn))

        dx = px - tx
        dy = py - ty
        dw = pw - tw
        dh = ph - th
        loc = dx * dx + dy * dy + dw * dw + dh * dh

        phw = pw * 0.5
        phh = ph * 0.5
        thw = tw * 0.5
        thh = th * 0.5
        iw = jnp.maximum(
            jnp.minimum(px + phw, tx + thw)
            - jnp.maximum(px - phw, tx - thw), 0.0)
        ih = jnp.maximum(
            jnp.minimum(py + phh, ty + thh)
            - jnp.maximum(py - phh, ty - thh), 0.0)
        inter = iw * ih
        union = pw * ph + tw * th - inter
        iou = inter / (union + 1e-6)

        dob = iou - pc
        dno = pc - tc
        obj = tc > 0.0
        nob = tc == 0.0
        return (jnp.where(obj, loc, 0.0),
                jnp.where(obj, dob * dob, 0.0),
                jnp.where(nob, dno * dno, 0.0))

    def lt_body(lt, carry):
        a_loc, a_co, a_cn = carry
        l0 = pl.multiple_of(lt * 128, 128)
        c_loc, c_co, c_cn = cell_terms(l0, 128)
        return a_loc + c_loc, a_co + c_co, a_cn + c_cn

    z = jnp.zeros((s2n, 128), jnp.float32)
    a_loc, a_co, a_cn = jax.lax.fori_loop(0, n_full, lt_body, (z, z, z),
                                          unroll=unroll)
    loc = jnp.sum(a_loc)
    co = jnp.sum(a_co)
    cn_ = jnp.sum(a_cn)
    if n_tail:
        t_loc, t_co, t_cn = cell_terms(n_full * 128, n_tail)
        loc = loc + jnp.sum(t_loc)
        co = co + jnp.sum(t_co)
        cn_ = cn_ + jnp.sum(t_cn)

    loc = lambda_coord * loc
    cn_ = lambda_noobj * cn_
    tot = loc + co + cn_
    ol = jax.lax.broadcasted_iota(jnp.int32, (8, 128), 1)
    out_ref[0] = jnp.where(ol == 0, tot,
                 jnp.where(ol == 1, loc,
                 jnp.where(ol == 2, co,
                 jnp.where(ol == 3, cn_, 0.0))))


@functools.partial(jax.jit, static_argnames=("lambda_coord", "lambda_noobj"))
def _yolo_v1_loss(pred, target, lambda_coord=5.0, lambda_noobj=0.5):
    assert pred.shape == target.shape and pred.shape[-1] == 5
    b, s1, s2, c = pred.shape

    pt = jnp.transpose(pred, (1, 3, 2, 0))
    tt = jnp.transpose(target, (1, 3, 2, 0))

    itemsize = np.dtype(pred.dtype).itemsize
    cost = pl.CostEstimate(
        flops=64 * b * s1 * s2 * c,
        transcendentals=0,
        bytes_accessed=2 * b * s1 * s2 * c * itemsize + s1 * 8 * 128 * 4)

    body = functools.partial(
        _yolo_kernel,
        lambda_coord=float(lambda_coord),
        lambda_noobj=float(lambda_noobj),
        n_b=b, unroll=4)

    out = pl.pallas_call(
        body,
        out_shape=jax.ShapeDtypeStruct((s1, 8, 128), jnp.float32),
        grid=(s1,),
        in_specs=[
            pl.BlockSpec((1, c, s2, b), lambda k: (k, 0, 0, 0)),
            pl.BlockSpec((1, c, s2, b), lambda k: (k, 0, 0, 0)),
        ],
        out_specs=pl.BlockSpec((1, 8, 128), lambda k: (k, 0, 0)),
        compiler_params=pltpu.CompilerParams(
            dimension_semantics=("parallel",),
            vmem_limit_bytes=48 * 1024 * 1024),
        cost_estimate=cost,
    )(pt, tt)

    totals = jnp.sum(out[:, 0, :4], axis=0)
    return totals[0], totals[1], totals[2], totals[3]


def kernel(pred, target):
    return _yolo_v1_loss(pred, target)
